# baseline (speedup 1.0000x reference)
import dataclasses

import jax
import jax.numpy as jnp
from jax import lax
from jax.experimental import pallas as pl
from jax.experimental.pallas import tpu as pltpu
from jax.experimental.pallas import tpu_sc as plsc


_NW = 32
_G = 4
_W = 50 * _G
_SB = 16
_SBR = _G * _SB
_SBW = _W * _SB


def _sc_embed(w4, rows2d, colb2d, batch, seq, dim):
    mesh = plsc.VectorSubcoreMesh(core_axis_name="c", subcore_axis_name="s")
    rows_per_worker = batch // _NW
    n_sb = rows_per_worker // _SBR

    cp = pltpu.CompilerParams()
    if "needs_layout_passes" in pltpu.CompilerParams.__dataclass_fields__:
        cp = dataclasses.replace(cp, needs_layout_passes=False)

    @pl.kernel(
        out_type=jax.ShapeDtypeStruct((batch, seq, dim), jnp.float32),
        mesh=mesh,
        compiler_params=cp,
        scratch_types=[
            pltpu.VMEM((_SBR, seq), jnp.int32),
            pltpu.VMEM((_SBW,), jnp.int32),
            pltpu.VMEM((_SBR, seq), jnp.int32),
            pltpu.VMEM((_SBW,), jnp.int32),
            pltpu.VMEM((_W, 128), jnp.float32),
            pltpu.VMEM((_W, 128), jnp.float32),
            pltpu.VMEM((_W, dim), jnp.float32),
            pltpu.VMEM((_W, dim), jnp.float32),
            pltpu.SemaphoreType.DMA,
            pltpu.SemaphoreType.DMA,
            pltpu.SemaphoreType.DMA,
            pltpu.SemaphoreType.DMA,
        ],
    )
    def k(w4_hbm, rows_hbm, colb_hbm, out_hbm, rslab, rlist, cbv, cbflat,
          fa, fb, oa, ob, gsa, gsb, ssa, ssb):
        wid = lax.axis_index("s") * 2 + lax.axis_index("c")
        row_base = wid * rows_per_worker

        def gather(c, fbuf, gsem):
            return pltpu.make_async_copy(
                w4_hbm.at[rlist.at[pl.ds(c * _W, _W)]], fbuf, gsem
            )

        def stores(s, c, obuf, ssem):
            dst_row = row_base + (s * _SB + c) * _G
            return [
                pltpu.make_async_copy(
                    obuf.at[pl.ds(gg * seq, seq)],
                    out_hbm.at[dst_row + gg],
                    ssem,
                )
                for gg in range(_G)
            ]

        def extract(c, fbuf, obuf):
            @pl.loop(0, _W // 16)
            def _(g):
                vec = cbflat[pl.ds(c * _W + g * 16, 16)]
                for k in range(16):
                    j = g * 16 + k
                    cb = vec[k]
                    obuf[j, pl.ds(0, 16)] = fbuf[j, pl.ds(cb, 16)]
                    obuf[j, pl.ds(16, 16)] = fbuf[j, pl.ds(cb + 16, 16)]

            gt = _W - 16
            vec = cbflat[pl.ds(c * _W + gt, 16)]
            for k in range(16):
                j = gt + k
                cb = vec[k]
                obuf[j, pl.ds(0, 16)] = fbuf[j, pl.ds(cb, 16)]
                obuf[j, pl.ds(16, 16)] = fbuf[j, pl.ds(cb + 16, 16)]

        def do_half(s, c, fbuf, obuf, gsem, ssem, first_thresh):
            gather(c, fbuf, gsem).wait()
            gc = s * _SB + c

            @pl.when(gc >= first_thresh)
            def _():
                for cp_ in stores(0, 0, obuf, ssem):
                    cp_.wait()

            extract(c, fbuf, obuf)
            for cp_ in stores(s, c, obuf, ssem):
                cp_.start()

        @pl.loop(0, n_sb)
        def _(s):
            sb_row = row_base + s * _SBR
            pltpu.sync_copy(rows_hbm.at[pl.ds(sb_row, _SBR)], rslab)
            pltpu.sync_copy(colb_hbm.at[pl.ds(sb_row, _SBR)], cbv)

            @pl.loop(0, _SBR)
            def _(r):
                for off in (0, 16, 32, seq - 16):
                    rlist[pl.ds(r * seq + off, 16)] = rslab[r, pl.ds(off, 16)]
                    cbflat[pl.ds(r * seq + off, 16)] = cbv[r, pl.ds(off, 16)]

            gather(0, fa, gsa).start()

            @pl.loop(0, _SB // 2)
            def _(it):
                c0 = 2 * it
                gather(c0 + 1, fb, gsb).start()
                do_half(s, c0, fa, oa, gsa, ssa, 2)

                @pl.when(it < _SB // 2 - 1)
                def _():
                    gather(c0 + 2, fa, gsa).start()

                do_half(s, c0 + 1, fb, ob, gsb, ssb, 3)

        for cp_ in stores(0, 0, oa, ssa):
            cp_.wait()
        for cp_ in stores(0, 0, ob, ssb):
            cp_.wait()

    return k(w4, rows2d, colb2d)


def kernel(token_ids, weight):
    b, s = token_ids.shape
    dim = weight.shape[1]
    ids = token_ids.astype(jnp.int32)
    rows2d = ids >> 2
    colb2d = (ids & 3) << 5
    w4 = weight.reshape(-1, 128)
    return _sc_embed(w4, rows2d, colb2d, b, s, dim)
